# baseline (speedup 1.0000x reference)
import functools

import jax
import jax.numpy as jnp
from jax import lax
from jax.experimental import pallas as pl
from jax.experimental.pallas import tpu as pltpu
from jax.experimental.pallas import tpu_sc as plsc


def _sc_window_copy(table, Q, K, offset):
    D = table.shape[1]
    info = plsc.get_sparse_core_info()
    NC = info.num_cores
    mesh = plsc.VectorSubcoreMesh(core_axis_name="c", subcore_axis_name="s")

    table_flat = table.reshape(-1)

    nchunks = 8
    chunk = (K * D) // nchunks

    @functools.partial(
        pl.kernel,
        out_type=jax.ShapeDtypeStruct((Q, K * D), table.dtype),
        mesh=mesh,
        scratch_types=[pltpu.SemaphoreType.DMA],
    )
    def copy_kernel(table_hbm, out_hbm, sem):
        wid = lax.axis_index("s") * NC + lax.axis_index("c")
        start = (offset - wid) * D
        for c in range(nchunks):
            pltpu.async_copy(
                table_hbm.at[pl.ds(start + c * chunk, chunk)],
                out_hbm.at[wid, pl.ds(c * chunk, chunk)],
                sem,
            )
        pltpu.make_async_copy(
            table_hbm.at[pl.ds(start, K * D)], out_hbm.at[wid], sem
        ).wait()

    return copy_kernel(table_flat).reshape(Q, K, D)


def kernel(q, k, embed_weight):
    Q = q.shape[0]
    K = k.shape[0]
    max_len = embed_weight.shape[0]
    offset = max_len // 2 + max_len % 2
    return _sc_window_copy(embed_weight, Q, K, offset)


# speedup vs baseline: 21.8939x; 21.8939x over previous
import functools

import jax
import jax.numpy as jnp
from jax import lax
from jax.experimental import pallas as pl
from jax.experimental.pallas import tpu as pltpu
from jax.experimental.pallas import tpu_sc as plsc


def _sc_window_copy(table, Q, K, offset):
    D = table.shape[1]
    info = plsc.get_sparse_core_info()
    NC = info.num_cores
    mesh = plsc.VectorSubcoreMesh(core_axis_name="c", subcore_axis_name="s")

    table_flat = table.reshape(-1)

    NBUF = 3
    nchunks = 16
    chunk = (K * D) // nchunks

    @functools.partial(
        pl.kernel,
        out_type=jax.ShapeDtypeStruct((Q * K * D,), table.dtype),
        mesh=mesh,
        scratch_types=[
            pltpu.VMEM((NBUF * chunk,), table.dtype),
            pltpu.SemaphoreType.DMA,
            pltpu.SemaphoreType.DMA,
        ],
    )
    def copy_kernel(table_hbm, out_hbm, buf, in_sem, out_sem):
        wid = lax.axis_index("s") * NC + lax.axis_index("c")
        start = (offset - wid) * D

        def fire_in(c):
            return pltpu.async_copy(
                table_hbm.at[pl.ds(start + c * chunk, chunk)],
                buf.at[pl.ds((c % NBUF) * chunk, chunk)],
                in_sem,
            )

        def fire_out(c):
            return pltpu.async_copy(
                buf.at[pl.ds((c % NBUF) * chunk, chunk)],
                out_hbm.at[pl.ds(wid * (K * D) + c * chunk, chunk)],
                out_sem,
            )

        ins = [None] * nchunks
        outs = [None] * nchunks
        for c in range(NBUF - 1):
            ins[c] = fire_in(c)
        for c in range(nchunks):
            nxt = c + NBUF - 1
            if nxt < nchunks:
                if c > 0:
                    outs[c - 1].wait()
                ins[nxt] = fire_in(nxt)
            ins[c].wait()
            outs[c] = fire_out(c)
        for c in range(nchunks - NBUF, nchunks):
            outs[c].wait()

    return copy_kernel(table_flat).reshape(Q, K, D)


def kernel(q, k, embed_weight):
    Q = q.shape[0]
    K = k.shape[0]
    max_len = embed_weight.shape[0]
    offset = max_len // 2 + max_len % 2
    return _sc_window_copy(embed_weight, Q, K, offset)


# speedup vs baseline: 50.8167x; 2.3210x over previous
import functools

import jax
import jax.numpy as jnp
from jax import lax
from jax.experimental import pallas as pl
from jax.experimental.pallas import tpu as pltpu
from jax.experimental.pallas import tpu_sc as plsc


def _sc_window_copy(table, Q, K, offset):
    D = table.shape[1]
    info = plsc.get_sparse_core_info()
    NC = info.num_cores
    NW = info.num_cores * info.num_subcores
    mesh = plsc.VectorSubcoreMesh(core_axis_name="c", subcore_axis_name="s")

    cpt = K // NW
    span = cpt + Q - 1
    span_al = ((span + 7) // 8) * 8

    table_flat = table.reshape(-1)

    @functools.partial(
        pl.kernel,
        out_type=jax.ShapeDtypeStruct((Q * K * D,), table.dtype),
        mesh=mesh,
        scratch_types=[
            pltpu.VMEM((span_al * D,), table.dtype),
            pltpu.SemaphoreType.DMA,
            pltpu.SemaphoreType.DMA,
        ],
    )
    def copy_kernel(table_hbm, out_hbm, buf, in_sem, out_sem):
        wid = lax.axis_index("s") * NC + lax.axis_index("c")
        block = wid * cpt
        lo = block + offset - (Q - 1)
        lo_al = (lo // 8) * 8

        pltpu.async_copy(
            table_hbm.at[pl.ds(lo_al * D, span_al * D)], buf, in_sem
        ).wait()

        outs = []
        for i in range(Q):
            src_off = (block + offset - i - lo_al) * D
            dst_off = (i * K + block) * D
            outs.append(
                pltpu.async_copy(
                    buf.at[pl.ds(src_off, cpt * D)],
                    out_hbm.at[pl.ds(dst_off, cpt * D)],
                    out_sem,
                )
            )
        for h in outs:
            h.wait()

    return copy_kernel(table_flat).reshape(Q, K, D)


def kernel(q, k, embed_weight):
    Q = q.shape[0]
    K = k.shape[0]
    max_len = embed_weight.shape[0]
    offset = max_len // 2 + max_len % 2
    return _sc_window_copy(embed_weight, Q, K, offset)
